# baseline (speedup 1.0000x reference)
import jax
import jax.numpy as jnp
from jax.experimental import pallas as pl

_BN = 2000


def _gates_body(zl_ref, zh_ref, w1t_ref, b1_ref, w2t_ref, b2_ref,
                ol_ref, oh_ref):
    ol_ref[...] = (
        jnp.dot(zl_ref[...], w1t_ref[...], preferred_element_type=jnp.float32)
        + b1_ref[...]
    )
    oh_ref[...] = (
        jnp.dot(zh_ref[...], w2t_ref[...], preferred_element_type=jnp.float32)
        + b2_ref[...]
    )


@jax.jit
def kernel(z_lp, z_hp, W1, b1, W2, b2):
    n, d = z_lp.shape
    w1t = W1.T
    w2t = W2.T
    b1r = b1.reshape(1, 2)
    b2r = b2.reshape(1, 2)
    grid = (n // _BN,)
    out_shape = (
        jax.ShapeDtypeStruct((n, 2), jnp.float32),
        jax.ShapeDtypeStruct((n, 2), jnp.float32),
    )
    x_lp, x_hp = pl.pallas_call(
        _gates_body,
        grid=grid,
        in_specs=[
            pl.BlockSpec((_BN, d), lambda i: (i, 0)),
            pl.BlockSpec((_BN, d), lambda i: (i, 0)),
            pl.BlockSpec((d, 2), lambda i: (0, 0)),
            pl.BlockSpec((1, 2), lambda i: (0, 0)),
            pl.BlockSpec((d, 2), lambda i: (0, 0)),
            pl.BlockSpec((1, 2), lambda i: (0, 0)),
        ],
        out_specs=(
            pl.BlockSpec((_BN, 2), lambda i: (i, 0)),
            pl.BlockSpec((_BN, 2), lambda i: (i, 0)),
        ),
        out_shape=out_shape,
    )(z_lp, z_hp, w1t, b1r, w2t, b2r)
    return (x_lp, x_hp)
